# Initial kernel scaffold; baseline (speedup 1.0000x reference)
import jax
import jax.numpy as jnp
from jax.experimental import pallas as pl


def kernel(edge_src, switch, species):
    raise NotImplementedError("write your pallas kernel here")


# speedup vs baseline: 29.9357x; 29.9357x over previous
import functools

import jax
import jax.numpy as jnp
from jax import lax
from jax.experimental import pallas as pl
from jax.experimental.pallas import tpu as pltpu
from jax.experimental.pallas import tpu_sc as plsc

_NC = 2
_NS = 16
_CHUNK = 25000


@functools.lru_cache(maxsize=None)
def _make_sc_segsum(n_edges: int, n_nodes: int, chunk: int):
    n_workers = _NC * _NS
    e_per_w = n_edges // n_workers
    assert e_per_w * n_workers == n_edges
    n_chunks = e_per_w // chunk
    assert n_chunks * chunk == e_per_w
    assert chunk % 8 == 0 and e_per_w % 8 == 0

    mesh = plsc.VectorSubcoreMesh(core_axis_name="c", subcore_axis_name="s")

    @functools.partial(
        pl.kernel,
        mesh=mesh,
        out_type=jax.ShapeDtypeStruct((_NC, n_nodes), jnp.float32),
        scratch_types=[
            pltpu.VMEM((chunk,), jnp.int32),
            pltpu.VMEM((chunk,), jnp.float32),
            pltpu.VMEM_SHARED((n_nodes,), jnp.float32),
        ],
    )
    def segsum(edge_src_hbm, vals_hbm, zeros_hbm, out_hbm, idx_v, val_v, acc):
        cid = lax.axis_index("c")
        sid = lax.axis_index("s")
        wid = sid * _NC + cid

        @pl.when(sid == 0)
        def _():
            pltpu.sync_copy(zeros_hbm, acc)

        plsc.subcore_barrier()

        def body(j, carry):
            base = wid * e_per_w + j * chunk
            pltpu.sync_copy(edge_src_hbm.at[pl.ds(base, chunk)], idx_v)
            pltpu.sync_copy(vals_hbm.at[pl.ds(base, chunk)], val_v)
            pltpu.sync_copy(val_v, acc.at[idx_v], add=True)
            return carry

        lax.fori_loop(0, n_chunks, body, 0)
        plsc.subcore_barrier()

        @pl.when(sid == 0)
        def _():
            pltpu.sync_copy(acc, out_hbm.at[cid])

    return segsum


def kernel(edge_src, switch, species):
    n_edges = edge_src.shape[0]
    n_nodes = species.shape[0]
    seg = _make_sc_segsum(n_edges, n_nodes, _CHUNK)
    zeros = jnp.zeros((n_nodes,), jnp.float32)
    partials = seg(edge_src, switch, zeros)
    return partials[0] + partials[1]


# speedup vs baseline: 33.0088x; 1.1027x over previous
import functools

import jax
import jax.numpy as jnp
from jax import lax
from jax.experimental import pallas as pl
from jax.experimental.pallas import tpu as pltpu
from jax.experimental.pallas import tpu_sc as plsc

_NC = 2
_NS = 16
_LANES = 16
_CHUNK = 10000


@functools.lru_cache(maxsize=None)
def _make_sc_segsum(n_edges: int, n_nodes: int, chunk: int):
    n_workers = _NC * _NS
    e_per_w = n_edges // n_workers
    n_chunks = e_per_w // chunk
    assert e_per_w * n_workers == n_edges
    assert n_chunks * chunk == e_per_w
    assert chunk % _LANES == 0 and chunk % 8 == 0 and e_per_w % 8 == 0

    seg = -(-n_nodes // (_NS * 8)) * 8
    n_pad = seg * _NS
    assert chunk >= seg

    mesh = plsc.VectorSubcoreMesh(core_axis_name="c", subcore_axis_name="s")

    @functools.partial(
        pl.kernel,
        mesh=mesh,
        out_type=(
            jax.ShapeDtypeStruct((n_workers * n_pad,), jnp.float32),
            jax.ShapeDtypeStruct((_NC * n_pad,), jnp.float32),
        ),
        scratch_types=[
            pltpu.VMEM((chunk + _LANES,), jnp.int32),
            pltpu.VMEM((chunk,), jnp.float32),
            pltpu.VMEM((n_pad,), jnp.float32),
            pltpu.SemaphoreType.DMA,
        ],
        compiler_params=pltpu.CompilerParams(needs_layout_passes=False),
    )
    def segsum(edge_src_hbm, vals_hbm, zeros_hbm, stage_hbm, out_hbm,
               idx_v, val_v, acc, sem):
        cid = lax.axis_index("c")
        sid = lax.axis_index("s")
        wid = cid * _NS + sid

        pltpu.sync_copy(zeros_hbm.at[pl.ds(wid * n_pad, n_pad)], acc)

        lane = lax.iota(jnp.int32, _LANES)
        m15 = lane == (_LANES - 1)

        def chunk_body(j, carry):
            base = wid * e_per_w + j * chunk
            pltpu.sync_copy(edge_src_hbm.at[pl.ds(base, chunk)],
                            idx_v.at[pl.ds(0, chunk)])
            pltpu.sync_copy(vals_hbm.at[pl.ds(base, chunk)], val_v)

            def vreg_body(i, c):
                o = i * _LANES
                idx = idx_v[pl.ds(o, _LANES)]
                nxt = idx_v[pl.ds(o + 1, _LANES)]
                val = val_v[pl.ds(o, _LANES)]
                s = plsc.cumsum(val)
                mb = idx != nxt
                plsc.addupdate_scatter(acc, [idx], s, mask=mb | m15)
                plsc.addupdate_scatter(acc, [nxt], -s, mask=mb & ~m15)
                return c

            lax.fori_loop(0, chunk // _LANES, vreg_body, 0)
            return carry

        lax.fori_loop(0, n_chunks, chunk_body, 0)

        pltpu.sync_copy(acc, stage_hbm.at[pl.ds(wid * n_pad, n_pad)])
        plsc.subcore_barrier()

        col = sid * seg
        copies = [
            pltpu.async_copy(
                stage_hbm.at[pl.ds((cid * _NS + t) * n_pad + col, seg)],
                acc.at[pl.ds(t * seg, seg)], sem)
            for t in range(_NS)
        ]
        for c in copies:
            c.wait()

        def merge_body(i, carry):
            o = i * _LANES
            tot = acc[pl.ds(o, _LANES)]
            for t in range(1, _NS):
                tot = tot + acc[pl.ds(t * seg + o, _LANES)]
            val_v[pl.ds(o, _LANES)] = tot
            return carry

        lax.fori_loop(0, seg // _LANES, merge_body, 0)
        pltpu.sync_copy(val_v.at[pl.ds(0, seg)],
                        out_hbm.at[pl.ds(cid * n_pad + col, seg)])

    return segsum, n_pad


def kernel(edge_src, switch, species):
    n_edges = edge_src.shape[0]
    n_nodes = species.shape[0]
    seg, n_pad = _make_sc_segsum(n_edges, n_nodes, _CHUNK)
    zeros = jnp.zeros((_NC * _NS * n_pad,), jnp.float32)
    _, partials = seg(edge_src, switch, zeros)
    partials = partials.reshape(_NC, n_pad)
    return (partials[0] + partials[1])[:n_nodes]


# speedup vs baseline: 59.9659x; 1.8167x over previous
import functools

import jax
import jax.numpy as jnp
from jax import lax
from jax.experimental import pallas as pl
from jax.experimental.pallas import tpu as pltpu
from jax.experimental.pallas import tpu_sc as plsc

_NC = 2
_NS = 16
_LANES = 16
_CHUNK = 10000


@functools.lru_cache(maxsize=None)
def _make_sc_segsum(n_edges: int, n_nodes: int, chunk: int):
    n_workers = _NC * _NS
    e_per_w = n_edges // n_workers
    n_chunks = e_per_w // chunk
    assert e_per_w * n_workers == n_edges
    assert n_chunks * chunk == e_per_w
    assert chunk % _LANES == 0 and chunk % 8 == 0 and e_per_w % 8 == 0

    seg = -(-n_nodes // (_NS * 8)) * 8
    n_pad = seg * _NS
    assert chunk >= seg

    mesh = plsc.VectorSubcoreMesh(core_axis_name="c", subcore_axis_name="s")

    @functools.partial(
        pl.kernel,
        mesh=mesh,
        out_type=(
            jax.ShapeDtypeStruct((n_workers * n_pad,), jnp.float32),
            jax.ShapeDtypeStruct((_NC * n_pad,), jnp.float32),
        ),
        scratch_types=[
            pltpu.VMEM((chunk + _LANES,), jnp.int32),
            pltpu.VMEM((chunk,), jnp.float32),
            pltpu.VMEM((n_pad,), jnp.float32),
            pltpu.SemaphoreType.DMA,
        ],
        compiler_params=pltpu.CompilerParams(needs_layout_passes=False),
    )
    def segsum(edge_src_hbm, vals_hbm, zeros_hbm, stage_hbm, out_hbm,
               idx_v, val_v, acc, sem):
        cid = lax.axis_index("c")
        sid = lax.axis_index("s")
        wid = cid * _NS + sid

        pltpu.sync_copy(zeros_hbm.at[pl.ds(wid * n_pad, n_pad)], acc)

        lane = lax.iota(jnp.int32, _LANES)
        m15 = lane == (_LANES - 1)

        def chunk_body(j, carry):
            base = wid * e_per_w + j * chunk
            pltpu.sync_copy(edge_src_hbm.at[pl.ds(base, chunk)],
                            idx_v.at[pl.ds(0, chunk)])
            pltpu.sync_copy(vals_hbm.at[pl.ds(base, chunk)], val_v)

            @plsc.parallel_loop(0, chunk, _LANES, unroll=8)
            def _(o):
                idx = idx_v[pl.ds(o, _LANES)]
                nxt = idx_v[pl.ds(o + 1, _LANES)]
                val = val_v[pl.ds(o, _LANES)]
                s = plsc.cumsum(val)
                mb = idx != nxt
                plsc.addupdate_scatter(acc, [idx], s, mask=mb | m15)
                plsc.addupdate_scatter(acc, [nxt], -s, mask=mb & ~m15)

            return carry

        lax.fori_loop(0, n_chunks, chunk_body, 0)

        pltpu.sync_copy(acc, stage_hbm.at[pl.ds(wid * n_pad, n_pad)])
        plsc.subcore_barrier()

        col = sid * seg
        copies = [
            pltpu.async_copy(
                stage_hbm.at[pl.ds((cid * _NS + t) * n_pad + col, seg)],
                acc.at[pl.ds(t * seg, seg)], sem)
            for t in range(_NS)
        ]
        for c in copies:
            c.wait()

        def merge_body(i, carry):
            o = i * _LANES
            tot = acc[pl.ds(o, _LANES)]
            for t in range(1, _NS):
                tot = tot + acc[pl.ds(t * seg + o, _LANES)]
            val_v[pl.ds(o, _LANES)] = tot
            return carry

        lax.fori_loop(0, seg // _LANES, merge_body, 0)
        pltpu.sync_copy(val_v.at[pl.ds(0, seg)],
                        out_hbm.at[pl.ds(cid * n_pad + col, seg)])

    return segsum, n_pad


def kernel(edge_src, switch, species):
    n_edges = edge_src.shape[0]
    n_nodes = species.shape[0]
    seg, n_pad = _make_sc_segsum(n_edges, n_nodes, _CHUNK)
    zeros = jnp.zeros((_NC * _NS * n_pad,), jnp.float32)
    _, partials = seg(edge_src, switch, zeros)
    partials = partials.reshape(_NC, n_pad)
    return (partials[0] + partials[1])[:n_nodes]
